# baseline (speedup 1.0000x reference)
import jax
import jax.numpy as jnp
from jax.experimental import pallas as pl
from jax.experimental.pallas import tpu as pltpu

_K = 9
_D = 2
_M = 625
_S = 36
_L = 128
_BLOCK_M = 25


def _deint_kernel(in_ref, out_ref):
    x = in_ref[...]
    idx = jax.lax.broadcasted_iota(jnp.int32, (x.shape[0], _S, _L // 2), 2) * 2
    out_ref[...] = jnp.take_along_axis(x, idx, axis=2)


def kernel(edge_index):
    two, b, n, kd = edge_index.shape
    flat = edge_index.reshape(_M, _S, _L)
    out = pl.pallas_call(
        _deint_kernel,
        grid=(_M // _BLOCK_M,),
        in_specs=[pl.BlockSpec((_BLOCK_M, _S, _L), lambda i: (i, 0, 0))],
        out_specs=pl.BlockSpec((_BLOCK_M, _S, _L // 2), lambda i: (i, 0, 0)),
        out_shape=jax.ShapeDtypeStruct((_M, _S, _L // 2), edge_index.dtype),
    )(flat)
    return out.reshape(two, b, n, _K)


# speedup vs baseline: 1.3765x; 1.3765x over previous
import jax
import jax.numpy as jnp
from jax.experimental import pallas as pl
from jax.experimental.pallas import tpu as pltpu

_K = 9
_D = 2


def _copy_kernel(in_ref, out_ref, sems):
    copies = [
        pltpu.make_async_copy(
            in_ref.at[:, :, pl.ds(_D * k, 1), :],
            out_ref.at[:, :, pl.ds(k, 1), :],
            sems.at[k],
        )
        for k in range(_K)
    ]
    for cp in copies:
        cp.start()
    for cp in copies:
        cp.wait()


def kernel(edge_index):
    two, b, n, kd = edge_index.shape
    t = jnp.transpose(edge_index, (0, 1, 3, 2))
    out_t = pl.pallas_call(
        _copy_kernel,
        in_specs=[pl.BlockSpec(memory_space=pl.ANY)],
        out_specs=pl.BlockSpec(memory_space=pl.ANY),
        out_shape=jax.ShapeDtypeStruct((two, b, _K, n), edge_index.dtype),
        scratch_shapes=[pltpu.SemaphoreType.DMA((_K,))],
    )(t)
    return jnp.transpose(out_t, (0, 1, 3, 2))


# speedup vs baseline: 1.7658x; 1.2828x over previous
import jax
import jax.numpy as jnp
from jax.experimental import pallas as pl
from jax.experimental.pallas import tpu as pltpu

_K = 9
_D = 2


def _copy_kernel(in_ref, out_ref, sems):
    copies = [
        pltpu.make_async_copy(
            in_ref.at[:, pl.ds(_D * k, 1), :, :],
            out_ref.at[:, pl.ds(k, 1), :, :],
            sems.at[k],
        )
        for k in range(_K)
    ]
    for cp in copies:
        cp.start()
    for cp in copies:
        cp.wait()


def kernel(edge_index):
    two, b, n, kd = edge_index.shape
    t = jnp.transpose(edge_index, (0, 3, 1, 2))
    out_t = pl.pallas_call(
        _copy_kernel,
        in_specs=[pl.BlockSpec(memory_space=pl.ANY)],
        out_specs=pl.BlockSpec(memory_space=pl.ANY),
        out_shape=jax.ShapeDtypeStruct((two, _K, b, n), edge_index.dtype),
        scratch_shapes=[pltpu.SemaphoreType.DMA((_K,))],
    )(t)
    return jnp.transpose(out_t, (0, 2, 3, 1))


# speedup vs baseline: 36.0676x; 20.4262x over previous
import jax
import jax.numpy as jnp
from jax.experimental import pallas as pl
from jax.experimental.pallas import tpu as pltpu

_K = 9
_D = 2


def _copy_kernel(in_ref, out_ref):
    out_ref[...] = in_ref[...]


def kernel(edge_index):
    two, b, n, kd = edge_index.shape
    t = jnp.transpose(edge_index, (0, 3, 1, 2))
    out_t = pl.pallas_call(
        _copy_kernel,
        grid=(_K,),
        in_specs=[pl.BlockSpec((two, 1, b, n), lambda k: (0, _D * k, 0, 0))],
        out_specs=pl.BlockSpec((two, 1, b, n), lambda k: (0, k, 0, 0)),
        out_shape=jax.ShapeDtypeStruct((two, _K, b, n), edge_index.dtype),
    )(t)
    return jnp.transpose(out_t, (0, 2, 3, 1))


# speedup vs baseline: 68.0972x; 1.8880x over previous
import jax
import jax.numpy as jnp
from jax.experimental import pallas as pl
from jax.experimental.pallas import tpu as pltpu

_K = 9
_D = 2


def _copy_kernel(*refs):
    out_ref = refs[_K]
    for k in range(_K):
        out_ref[:, k : k + 1, :, :] = refs[k][...]


def _slab_spec(k, b, n):
    return pl.BlockSpec((1, 1, b, n), lambda i, _k=k: (i, _D * _k, 0, 0))


def kernel(edge_index):
    two, b, n, kd = edge_index.shape
    t = jnp.transpose(edge_index, (0, 3, 1, 2))
    out_t = pl.pallas_call(
        _copy_kernel,
        grid=(two,),
        in_specs=[_slab_spec(k, b, n) for k in range(_K)],
        out_specs=pl.BlockSpec((1, _K, b, n), lambda i: (i, 0, 0, 0)),
        out_shape=jax.ShapeDtypeStruct((two, _K, b, n), edge_index.dtype),
    )(*([t] * _K))
    return jnp.transpose(out_t, (0, 2, 3, 1))
